# baseline (speedup 1.0000x reference)
import functools
import math

import jax
import jax.numpy as jnp
import numpy as np
from jax import lax
from jax.experimental import pallas as pl
from jax.experimental.pallas import tpu as pltpu
from jax.experimental.pallas import tpu_sc as plsc

NUM_EMBED = 1000000
EMBED_DIM = 64
SEQ_LEN = 200
BATCH = 4096
BL = BATCH * SEQ_LEN

NC, NS = 2, 16
NW = NC * NS
PER_W = BL // NW
CHUNK = 128
NCHUNK = PER_W // CHUNK
LANES = 16
DSUB = EMBED_DIM // LANES


def _pe2() -> np.ndarray:
    position = np.arange(0, SEQ_LEN, dtype=np.float32)[:, None]
    div_term = np.exp(
        np.arange(0, EMBED_DIM, 2, dtype=np.float32)
        * (-math.log(10000.0) / EMBED_DIM)
    )
    pe = np.zeros((SEQ_LEN, EMBED_DIM), dtype=np.float32)
    pe[:, 0::2] = np.sin(position * div_term)
    pe[:, 1::2] = np.cos(position * div_term)
    return np.concatenate([pe, pe], axis=0)


@functools.partial(
    pl.kernel,
    out_type=jax.ShapeDtypeStruct((BL, EMBED_DIM), jnp.float32),
    mesh=plsc.VectorSubcoreMesh(
        core_axis_name="c", subcore_axis_name="s", num_cores=NC, num_subcores=NS
    ),
    scratch_types=[
        pltpu.VMEM((PER_W,), jnp.int32),
        pltpu.VMEM((2 * SEQ_LEN, EMBED_DIM), jnp.float32),
        pltpu.VMEM((CHUNK, EMBED_DIM), jnp.float32),
        pltpu.SemaphoreType.DMA,
    ],
    compiler_params=pltpu.CompilerParams(use_tc_tiling_on_sc=False),
)
def _sc_embed(x_hbm, table_hbm, pe_hbm, out_hbm, idx_v, pe_v, rows_v, gsem):
    wid = lax.axis_index("s") * NC + lax.axis_index("c")
    base = wid * PER_W
    pltpu.sync_copy(x_hbm.at[pl.ds(base, PER_W)], idx_v)
    pltpu.sync_copy(pe_hbm, pe_v)

    def chunk_body(c, carry):
        phase = lax.rem(c * CHUNK, SEQ_LEN)
        rbase = base + c * CHUNK
        pltpu.async_copy(
            table_hbm.at[idx_v.at[pl.ds(c * CHUNK, CHUNK)]], rows_v, gsem
        ).wait()

        def add_body(i, acc):
            pos = phase + i
            for k in range(DSUB):
                sl = pl.ds(k * LANES, LANES)
                rows_v[i, sl] = rows_v[i, sl] + pe_v[pos, sl]
            return acc

        lax.fori_loop(0, CHUNK, add_body, 0)
        pltpu.sync_copy(rows_v, out_hbm.at[pl.ds(rbase, CHUNK)])
        return carry

    lax.fori_loop(0, NCHUNK, chunk_body, 0)


def kernel(x, table):
    pe2 = jnp.asarray(_pe2())
    out = _sc_embed(x.reshape(-1), table, pe2)
    return out.reshape(BATCH, SEQ_LEN, EMBED_DIM)


# speedup vs baseline: 1.5355x; 1.5355x over previous
import functools
import math

import jax
import jax.numpy as jnp
import numpy as np
from jax import lax
from jax.experimental import pallas as pl
from jax.experimental.pallas import tpu as pltpu
from jax.experimental.pallas import tpu_sc as plsc

NUM_EMBED = 1000000
EMBED_DIM = 64
SEQ_LEN = 200
BATCH = 4096
BL = BATCH * SEQ_LEN

NC, NS = 2, 16
NW = NC * NS
PER_W = BL // NW
CHUNK = 128
NCHUNK = PER_W // CHUNK
LANES = 16
DSUB = EMBED_DIM // LANES
NBUF = 4
LOOKAHEAD = NBUF - 1


def _pe2() -> np.ndarray:
    position = np.arange(0, SEQ_LEN, dtype=np.float32)[:, None]
    div_term = np.exp(
        np.arange(0, EMBED_DIM, 2, dtype=np.float32)
        * (-math.log(10000.0) / EMBED_DIM)
    )
    pe = np.zeros((SEQ_LEN, EMBED_DIM), dtype=np.float32)
    pe[:, 0::2] = np.sin(position * div_term)
    pe[:, 1::2] = np.cos(position * div_term)
    return np.concatenate([pe, pe], axis=0)


@functools.partial(
    pl.kernel,
    out_type=jax.ShapeDtypeStruct((BL, EMBED_DIM), jnp.float32),
    mesh=plsc.VectorSubcoreMesh(
        core_axis_name="c", subcore_axis_name="s", num_cores=NC, num_subcores=NS
    ),
    scratch_types=[
        pltpu.VMEM((PER_W,), jnp.int32),
        pltpu.VMEM((2 * SEQ_LEN, EMBED_DIM), jnp.float32),
    ]
    + [pltpu.VMEM((CHUNK, EMBED_DIM), jnp.float32) for _ in range(NBUF)]
    + [pltpu.SemaphoreType.DMA for _ in range(2 * NBUF)],
    compiler_params=pltpu.CompilerParams(use_tc_tiling_on_sc=False),
)
def _sc_embed(x_hbm, table_hbm, pe_hbm, out_hbm, idx_v, pe_v, *bufs):
    rows = bufs[:NBUF]
    gsems = bufs[NBUF : 2 * NBUF]
    osems = bufs[2 * NBUF :]

    wid = lax.axis_index("s") * NC + lax.axis_index("c")
    base = wid * PER_W
    pltpu.sync_copy(x_hbm.at[pl.ds(base, PER_W)], idx_v)
    pltpu.sync_copy(pe_hbm, pe_v)

    def issue_gather(c, b):
        pltpu.async_copy(
            table_hbm.at[idx_v.at[pl.ds(c * CHUNK, CHUNK)]], rows[b], gsems[b]
        )

    def wait_gather(b):
        pltpu.make_async_copy(
            table_hbm.at[pl.ds(0, CHUNK)], rows[b], gsems[b]
        ).wait()

    def drain_out(b):
        pltpu.make_async_copy(
            rows[b], out_hbm.at[pl.ds(0, CHUNK)], osems[b]
        ).wait()

    for b in range(LOOKAHEAD):
        issue_gather(jnp.int32(b), b)

    def group_body(g4, carry):
        g = g4 * NBUF
        for b in range(NBUF):
            c = g + b
            wait_gather(b)

            phase = lax.rem(c * CHUNK, SEQ_LEN)

            @plsc.parallel_loop(0, CHUNK, 1, unroll=4)
            def _add(i):
                pos = phase + i
                for k in range(DSUB):
                    sl = pl.ds(k * LANES, LANES)
                    rows[b][i, sl] = rows[b][i, sl] + pe_v[pos, sl]

            pltpu.async_copy(
                rows[b], out_hbm.at[pl.ds(base + c * CHUNK, CHUNK)], osems[b]
            )

            nb = (b + LOOKAHEAD) % NBUF
            nxt = c + LOOKAHEAD

            @pl.when(jnp.logical_and(c >= 1, nxt < NCHUNK))
            def _():
                drain_out(nb)

            @pl.when(nxt < NCHUNK)
            def _():
                issue_gather(nxt, nb)

        return carry

    lax.fori_loop(0, NCHUNK // NBUF, group_body, 0)
    for b in range(NBUF):
        drain_out(b)


def kernel(x, table):
    pe2 = jnp.asarray(_pe2())
    out = _sc_embed(x.reshape(-1), table, pe2)
    return out.reshape(BATCH, SEQ_LEN, EMBED_DIM)
